# baseline (speedup 1.0000x reference)
import jax
import jax.numpy as jnp
from jax.experimental import pallas as pl
from jax.experimental.pallas import tpu as pltpu

E = 64
D = 768
DFF = 1536
LANE = 128
T = 2048
NTILES = T // LANE
GEXP0 = 1
GUN0 = 1 + E
GRID = 1 + E + NTILES


def _gelu_exact(x):
    return 0.5 * x * (1.0 + jax.lax.erf(x * 0.7071067811865476))


def _body(x_ref, wg_ref, bfc_ref, bproj_ref, wfc_ref, wproj_ref,
          logits_ref, aux_ref, outT_ref,
          xT_scr, sortedT_scr, poscol_scr, posrow_scr, meta_smem):
    g = pl.program_id(0)

    @pl.when(g == 0)
    def _():
        x = x_ref[...]
        xT_scr[...] = jnp.transpose(x)
        logits = jnp.dot(x, wg_ref[...], preferred_element_type=jnp.float32)
        logits_ref[...] = logits
        iota = jax.lax.broadcasted_iota(jnp.int32, (T, E), 1)
        rowmax = jnp.max(logits, axis=1, keepdims=True)
        ismax = logits == rowmax
        sel = jnp.min(jnp.where(ismax, iota, E), axis=1, keepdims=True)
        onehot = (iota == sel).astype(jnp.float32)
        counts = jnp.sum(onehot, axis=0, keepdims=True)
        ei = jax.lax.broadcasted_iota(jnp.int32, (E, E), 0)
        ej = jax.lax.broadcasted_iota(jnp.int32, (E, E), 1)
        ut = (ei < ej).astype(jnp.float32)
        starts_row = jnp.dot(counts, ut,
                             preferred_element_type=jnp.float32)
        counts_i = counts.astype(jnp.int32)
        starts_i = starts_row.astype(jnp.int32)
        for k in range(E):
            meta_smem[0, k] = starts_i[0, k]
            meta_smem[0, E + k] = counts_i[0, k]
        chunk = 256
        for ci in range(T // chunk):
            r_i = jax.lax.broadcasted_iota(jnp.int32, (chunk, T), 0)
            u_i = jax.lax.broadcasted_iota(jnp.int32, (chunk, T), 1)
            lc = (u_i < ci * chunk + r_i).astype(jnp.float32)
            rank = jnp.dot(lc, onehot, preferred_element_type=jnp.float32)
            slot = rank + starts_row
            oh_c = onehot[ci * chunk:(ci + 1) * chunk]
            pos_c = jnp.sum(slot * oh_c, axis=1, keepdims=True)
            pos_ci = pos_c.astype(jnp.int32)
            poscol_scr[ci * chunk:(ci + 1) * chunk, :] = pos_ci
            posrow_scr[:, ci * chunk:(ci + 1) * chunk] = jnp.transpose(pos_ci)
        ex = jnp.exp(logits - rowmax)
        sumex = jnp.sum(ex, axis=1, keepdims=True)
        acc = jnp.sum(ex / sumex, axis=0, keepdims=True)
        lse = rowmax + jnp.log(sumex)
        z = jnp.sum(lse * lse) / T
        acc_n = acc / jnp.maximum(jnp.sum(jnp.abs(acc)), 1e-12)
        freq_n = counts / jnp.maximum(jnp.sum(jnp.abs(counts)), 1e-12)
        switch = E * jnp.sum(acc_n * freq_n)
        aux_ref[...] = jnp.reshape(switch + 0.1 * z, (1, 1))

    @pl.when((g >= GEXP0) & (g < GUN0))
    def _():
        e = g - GEXP0
        s = meta_smem[0, e]
        c = meta_smem[0, E + e]
        end = s + c
        t0 = s // LANE
        nt = (end - t0 * LANE + LANE - 1) // LANE
        wfc = wfc_ref[...]
        wproj = wproj_ref[...]
        ohe = (jax.lax.broadcasted_iota(jnp.int32, (E, 1), 0) == e
               ).astype(jnp.float32)
        bfc = jnp.dot(bfc_ref[...], ohe, preferred_element_type=jnp.float32)
        bproj = jnp.dot(bproj_ref[...], ohe,
                        preferred_element_type=jnp.float32)

        def tile(j, carry):
            base = pl.multiple_of((t0 + j) * LANE, LANE)
            pcol = poscol_scr[...]
            lcol = base + jax.lax.broadcasted_iota(jnp.int32, (T, LANE), 1)
            ohg = (pcol == lcol).astype(jnp.float32)
            xt = jnp.dot(xT_scr[...], ohg,
                         preferred_element_type=jnp.float32)
            h = jnp.dot(wfc, xt, preferred_element_type=jnp.float32) + bfc
            h = _gelu_exact(h)
            o = jnp.dot(wproj, h, preferred_element_type=jnp.float32) + bproj
            col = base + jax.lax.broadcasted_iota(jnp.int32, (1, LANE), 1)
            m = (col >= s) & (col < end)
            sortedT_scr[:, pl.ds(base, LANE)] = jnp.where(
                m, o, sortedT_scr[:, pl.ds(base, LANE)])
            return carry

        @pl.when(c > 0)
        def _():
            jax.lax.fori_loop(0, nt, tile, 0)

    @pl.when(g >= GUN0)
    def _():
        tb = pl.multiple_of((g - GUN0) * LANE, LANE)
        prow = posrow_scr[:, pl.ds(tb, LANE)]
        jcol = jax.lax.broadcasted_iota(jnp.int32, (T, LANE), 0)
        oh2 = (jcol == prow).astype(jnp.float32)
        outT_ref[...] = jnp.dot(sortedT_scr[...], oh2,
                                preferred_element_type=jnp.float32)


def kernel(hidden_states, W_gate, W_fc, b_fc, W_proj, b_proj):
    b, s_len, _ = hidden_states.shape
    x = hidden_states.reshape(-1, D)
    t = x.shape[0]

    logits, aux, outT = pl.pallas_call(
        _body,
        grid=(GRID,),
        in_specs=[
            pl.BlockSpec((t, D), lambda g: (0, 0)),
            pl.BlockSpec((D, E), lambda g: (0, 0)),
            pl.BlockSpec((DFF, E), lambda g: (0, 0)),
            pl.BlockSpec((D, E), lambda g: (0, 0)),
            pl.BlockSpec((DFF, D),
                         lambda g: (0, jnp.clip(g - GEXP0, 0, E - 1))),
            pl.BlockSpec((D, DFF),
                         lambda g: (0, jnp.clip(g - GEXP0, 0, E - 1))),
        ],
        out_specs=(
            pl.BlockSpec((t, E), lambda g: (0, 0)),
            pl.BlockSpec((1, 1), lambda g: (0, 0)),
            pl.BlockSpec((D, LANE),
                         lambda g: (0, jnp.maximum(g - GUN0, 0))),
        ),
        scratch_shapes=[
            pltpu.VMEM((D, t), jnp.float32),
            pltpu.VMEM((D, t), jnp.float32),
            pltpu.VMEM((t, 1), jnp.int32),
            pltpu.VMEM((1, t), jnp.int32),
            pltpu.SMEM((1, 2 * E), jnp.int32),
        ],
        out_shape=(
            jax.ShapeDtypeStruct((t, E), jnp.float32),
            jax.ShapeDtypeStruct((1, 1), jnp.float32),
            jax.ShapeDtypeStruct((D, t), jnp.float32),
        ),
    )(x, W_gate, b_fc, b_proj,
      W_fc.reshape(DFF, E * D), W_proj.reshape(D, E * DFF))

    out = outT.T
    return out.reshape(b, s_len, D), logits, aux[0, 0]


# speedup vs baseline: 1.1584x; 1.1584x over previous
import jax
import jax.numpy as jnp
from jax.experimental import pallas as pl
from jax.experimental.pallas import tpu as pltpu

E = 64
D = 768
DFF = 1536
LANE = 128


def _gelu_exact(x):
    return 0.5 * x * (1.0 + jax.lax.erf(x * 0.7071067811865476))


import functools
from jax import lax
from jax.experimental.pallas import tpu_sc as plsc

_SC_NW = 32


def _sc_permute(to_sorted):
    mesh = plsc.VectorSubcoreMesh(core_axis_name="c", subcore_axis_name="s")
    t, bpw = 2048, 2048 // _SC_NW

    @functools.partial(
        pl.kernel, mesh=mesh,
        out_type=jax.ShapeDtypeStruct((t, D), jnp.float32),
        scratch_types=[
            pltpu.VMEM((bpw,), jnp.int32),
            pltpu.VMEM((bpw, D), jnp.float32),
            pltpu.SemaphoreType.DMA,
        ],
    )
    def k(rows_hbm, pos_hbm, out_hbm, idx_v, rows_v, sem):
        wid = lax.axis_index("s") * 2 + lax.axis_index("c")
        base = wid * bpw
        pltpu.sync_copy(pos_hbm.at[pl.ds(base, bpw)], idx_v)
        if to_sorted:
            pltpu.sync_copy(rows_hbm.at[pl.ds(base, bpw)], rows_v)
            pltpu.async_copy(rows_v, out_hbm.at[idx_v], sem).wait()
        else:
            pltpu.async_copy(rows_hbm.at[idx_v], rows_v, sem).wait()
            pltpu.sync_copy(rows_v, out_hbm.at[pl.ds(base, bpw)])

    return k


@functools.lru_cache(maxsize=None)
def _sc_permute_cached(to_sorted):
    return _sc_permute(to_sorted)


def _sc_to_sorted(rows, pos):
    return _sc_permute_cached(True)(rows, pos)


def _sc_from_sorted(rows, pos):
    return _sc_permute_cached(False)(rows, pos)


def _router_body(x_ref, wg_ref, logits_ref, pos_ref, counts_ref, aux_ref):
    x = x_ref[...]
    wg = wg_ref[...]
    logits = jnp.dot(x, wg, preferred_element_type=jnp.float32)
    logits_ref[...] = logits
    t = logits.shape[0]
    iota = jax.lax.broadcasted_iota(jnp.int32, (t, E), 1)
    rowmax = jnp.max(logits, axis=1, keepdims=True)
    ismax = logits == rowmax
    sel = jnp.min(jnp.where(ismax, iota, E), axis=1, keepdims=True)
    onehot = (iota == sel).astype(jnp.float32)
    counts = jnp.sum(onehot, axis=0, keepdims=True)
    counts_ref[...] = counts.astype(jnp.int32)
    ei = jax.lax.broadcasted_iota(jnp.int32, (E, E), 0)
    ej = jax.lax.broadcasted_iota(jnp.int32, (E, E), 1)
    ut = (ei < ej).astype(jnp.float32)
    starts_row = jnp.dot(counts, ut, preferred_element_type=jnp.float32)
    chunk = 256
    for ci in range(t // chunk):
        r_i = jax.lax.broadcasted_iota(jnp.int32, (chunk, t), 0)
        u_i = jax.lax.broadcasted_iota(jnp.int32, (chunk, t), 1)
        lc = (u_i < ci * chunk + r_i).astype(jnp.float32)
        rank = jnp.dot(lc, onehot, preferred_element_type=jnp.float32)
        slot = rank + starts_row
        oh_c = onehot[ci * chunk:(ci + 1) * chunk]
        pos_c = jnp.sum(slot * oh_c, axis=1, keepdims=True)
        pos_ref[ci * chunk:(ci + 1) * chunk, :] = pos_c.astype(jnp.int32)
    ex = jnp.exp(logits - rowmax)
    sumex = jnp.sum(ex, axis=1, keepdims=True)
    acc = jnp.sum(ex / sumex, axis=0, keepdims=True)
    lse = rowmax + jnp.log(sumex)
    z = jnp.sum(lse * lse) / t
    acc_n = acc / jnp.maximum(jnp.sum(jnp.abs(acc)), 1e-12)
    freq_n = counts / jnp.maximum(jnp.sum(jnp.abs(counts)), 1e-12)
    switch = E * jnp.sum(acc_n * freq_n)
    aux_ref[...] = jnp.reshape(switch + 0.1 * z, (1, 1))


def _ffn_body(ge_ref, gt_ref, starts_ref, counts_ref, xt_ref, wfc_ref,
              bfc_ref, wproj_ref, bproj_ref, out_ref):
    g = pl.program_id(0)
    e = ge_ref[g]
    s = starts_ref[e]
    c = counts_ref[e]
    base = gt_ref[g] * LANE
    xt = xt_ref[...]
    wfc = wfc_ref[...]
    wproj = wproj_ref[...]
    oh = (jax.lax.broadcasted_iota(jnp.int32, (E, 1), 0) == e
          ).astype(jnp.float32)
    bfc = jnp.dot(bfc_ref[...], oh, preferred_element_type=jnp.float32)
    bproj = jnp.dot(bproj_ref[...], oh, preferred_element_type=jnp.float32)
    h = jnp.dot(wfc, xt, preferred_element_type=jnp.float32) + bfc
    h = _gelu_exact(h)
    o = jnp.dot(wproj, h, preferred_element_type=jnp.float32) + bproj
    col = base + jax.lax.broadcasted_iota(jnp.int32, (1, LANE), 1)
    m = (col >= s) & (col < s + c)
    out_ref[...] = jnp.where(m, o, out_ref[...])


def kernel(hidden_states, W_gate, W_fc, b_fc, W_proj, b_proj):
    b, s_len, _ = hidden_states.shape
    x = hidden_states.reshape(-1, D)
    t = x.shape[0]
    num_tiles = t // LANE
    grid_sz = num_tiles + E - 1

    logits, pos, counts, aux = pl.pallas_call(
        _router_body,
        out_shape=(
            jax.ShapeDtypeStruct((t, E), jnp.float32),
            jax.ShapeDtypeStruct((t, 1), jnp.int32),
            jax.ShapeDtypeStruct((1, E), jnp.int32),
            jax.ShapeDtypeStruct((1, 1), jnp.float32),
        ),
    )(x, W_gate)

    pos1 = pos[:, 0]
    counts1 = counts[0]
    starts = (jnp.cumsum(counts1) - counts1).astype(jnp.int32)
    x_sorted = _sc_to_sorted(x, pos1)
    xT_sorted = x_sorted.T

    ends = starts + counts1
    ntiles = jnp.where(counts1 > 0,
                       (ends - 1) // LANE - starts // LANE + 1, 0)
    offs = jnp.cumsum(ntiles)
    total = offs[-1]
    gidx = jnp.arange(grid_sz, dtype=jnp.int32)
    ge = jnp.sum((offs[None, :] <= gidx[:, None]).astype(jnp.int32),
                 axis=1)
    ge = jnp.minimum(ge, E - 1)
    sel_mask = (ge[:, None] ==
                jnp.arange(E, dtype=jnp.int32)[None, :]).astype(jnp.int32)
    offs_ge = jnp.sum(sel_mask * offs[None, :], axis=1)
    ntiles_ge = jnp.sum(sel_mask * ntiles[None, :], axis=1)
    starts_ge = jnp.sum(sel_mask * starts[None, :], axis=1)
    first_of_e = offs_ge - ntiles_ge
    gt = starts_ge // LANE + (gidx - first_of_e)
    last = total - 1
    is_last = (gidx == last).astype(jnp.int32)
    ge_last = jnp.sum(is_last * ge)
    gt_last = jnp.sum(is_last * gt)
    ge = jnp.where(gidx <= last, ge, ge_last).astype(jnp.int32)
    gt = jnp.where(gidx <= last, gt, gt_last).astype(jnp.int32)

    grid_spec = pltpu.PrefetchScalarGridSpec(
        num_scalar_prefetch=4,
        grid=(grid_sz,),
        in_specs=[
            pl.BlockSpec((D, LANE),
                         lambda g, ge_r, gt_r, s_r, c_r: (0, gt_r[g])),
            pl.BlockSpec((DFF, D),
                         lambda g, ge_r, gt_r, s_r, c_r: (0, ge_r[g])),
            pl.BlockSpec((DFF, E),
                         lambda g, ge_r, gt_r, s_r, c_r: (0, 0)),
            pl.BlockSpec((D, DFF),
                         lambda g, ge_r, gt_r, s_r, c_r: (0, ge_r[g])),
            pl.BlockSpec((D, E),
                         lambda g, ge_r, gt_r, s_r, c_r: (0, 0)),
        ],
        out_specs=pl.BlockSpec((D, LANE),
                               lambda g, ge_r, gt_r, s_r, c_r: (0, gt_r[g])),
    )
    outT = pl.pallas_call(
        _ffn_body,
        grid_spec=grid_spec,
        out_shape=jax.ShapeDtypeStruct((D, t), jnp.float32),
    )(ge, gt, starts, counts1, xT_sorted,
      W_fc.reshape(DFF, E * D), b_fc,
      W_proj.reshape(D, E * DFF), b_proj)

    h_rows = outT.T
    out = _sc_from_sorted(h_rows, pos1)
    return out.reshape(b, s_len, D), logits, aux[0, 0]


# speedup vs baseline: 1.1826x; 1.0209x over previous
import jax
import jax.numpy as jnp
from jax.experimental import pallas as pl
from jax.experimental.pallas import tpu as pltpu

E = 64
D = 768
DFF = 1536
LANE = 128


def _gelu_exact(x):
    return 0.5 * x * (1.0 + jax.lax.erf(x * 0.7071067811865476))


import functools
from jax import lax
from jax.experimental.pallas import tpu_sc as plsc

_SC_NW = 32


def _sc_permute(to_sorted):
    mesh = plsc.VectorSubcoreMesh(core_axis_name="c", subcore_axis_name="s")
    t, bpw = 2048, 2048 // _SC_NW

    @functools.partial(
        pl.kernel, mesh=mesh,
        out_type=jax.ShapeDtypeStruct((t, D), jnp.float32),
        scratch_types=[
            pltpu.VMEM((bpw,), jnp.int32),
            pltpu.VMEM((bpw, D), jnp.float32),
            pltpu.SemaphoreType.DMA,
        ],
    )
    def k(rows_hbm, pos_hbm, out_hbm, idx_v, rows_v, sem):
        wid = lax.axis_index("s") * 2 + lax.axis_index("c")
        base = wid * bpw
        pltpu.sync_copy(pos_hbm.at[pl.ds(base, bpw)], idx_v)
        if to_sorted:
            pltpu.sync_copy(rows_hbm.at[pl.ds(base, bpw)], rows_v)
            pltpu.async_copy(rows_v, out_hbm.at[idx_v], sem).wait()
        else:
            pltpu.async_copy(rows_hbm.at[idx_v], rows_v, sem).wait()
            pltpu.sync_copy(rows_v, out_hbm.at[pl.ds(base, bpw)])

    return k


@functools.lru_cache(maxsize=None)
def _sc_permute_cached(to_sorted):
    return _sc_permute(to_sorted)


def _sc_to_sorted(rows, pos):
    return _sc_permute_cached(True)(rows, pos)


def _sc_from_sorted(rows, pos):
    return _sc_permute_cached(False)(rows, pos)


def _router_body(x_ref, wg_ref, logits_ref, pos_ref, counts_ref, aux_ref):
    x = x_ref[...]
    wg = wg_ref[...]
    logits = jnp.dot(x, wg, preferred_element_type=jnp.float32)
    logits_ref[...] = logits
    t = logits.shape[0]
    iota = jax.lax.broadcasted_iota(jnp.int32, (t, E), 1)
    rowmax = jnp.max(logits, axis=1, keepdims=True)
    ismax = logits == rowmax
    sel = jnp.min(jnp.where(ismax, iota, E), axis=1, keepdims=True)
    onehot = (iota == sel).astype(jnp.float32)
    counts = jnp.sum(onehot, axis=0, keepdims=True)
    counts_ref[...] = counts.astype(jnp.int32)
    ei = jax.lax.broadcasted_iota(jnp.int32, (E, E), 0)
    ej = jax.lax.broadcasted_iota(jnp.int32, (E, E), 1)
    ut = (ei < ej).astype(jnp.float32)
    starts_row = jnp.dot(counts, ut, preferred_element_type=jnp.float32)
    chunk = 256
    for ci in range(t // chunk):
        r_i = jax.lax.broadcasted_iota(jnp.int32, (chunk, t), 0)
        u_i = jax.lax.broadcasted_iota(jnp.int32, (chunk, t), 1)
        lc = (u_i < ci * chunk + r_i).astype(jnp.float32)
        rank = jnp.dot(lc, onehot, preferred_element_type=jnp.float32)
        slot = rank + starts_row
        oh_c = onehot[ci * chunk:(ci + 1) * chunk]
        pos_c = jnp.sum(slot * oh_c, axis=1, keepdims=True)
        pos_ref[ci * chunk:(ci + 1) * chunk, :] = pos_c.astype(jnp.int32)
    ex = jnp.exp(logits - rowmax)
    sumex = jnp.sum(ex, axis=1, keepdims=True)
    acc = jnp.sum(ex / sumex, axis=0, keepdims=True)
    lse = rowmax + jnp.log(sumex)
    z = jnp.sum(lse * lse) / t
    acc_n = acc / jnp.maximum(jnp.sum(jnp.abs(acc)), 1e-12)
    freq_n = counts / jnp.maximum(jnp.sum(jnp.abs(counts)), 1e-12)
    switch = E * jnp.sum(acc_n * freq_n)
    aux_ref[...] = jnp.reshape(switch + 0.1 * z, (1, 1))


def _ffn_body(ge_ref, gt_ref, starts_ref, counts_ref, xs_ref, wfc_ref,
              bfc_ref, wproj_ref, bproj_ref, out_ref, xT_scr, sortedT_scr):
    g = pl.program_id(0)
    ng = pl.num_programs(0)

    @pl.when(g == 0)
    def _():
        xT_scr[...] = jnp.transpose(xs_ref[...])

    e = ge_ref[g]
    s = starts_ref[e]
    c = counts_ref[e]
    base = pl.multiple_of(gt_ref[g] * LANE, LANE)
    xt = xT_scr[:, pl.ds(base, LANE)]
    wfc = wfc_ref[...]
    wproj = wproj_ref[...]
    oh = (jax.lax.broadcasted_iota(jnp.int32, (E, 1), 0) == e
          ).astype(jnp.float32)
    bfc = jnp.dot(bfc_ref[...], oh, preferred_element_type=jnp.float32)
    bproj = jnp.dot(bproj_ref[...], oh, preferred_element_type=jnp.float32)
    h = jnp.dot(wfc, xt, preferred_element_type=jnp.float32) + bfc
    h = _gelu_exact(h)
    o = jnp.dot(wproj, h, preferred_element_type=jnp.float32) + bproj
    col = base + jax.lax.broadcasted_iota(jnp.int32, (1, LANE), 1)
    m = (col >= s) & (col < s + c)
    sortedT_scr[:, pl.ds(base, LANE)] = jnp.where(
        m, o, sortedT_scr[:, pl.ds(base, LANE)])

    @pl.when(g == ng - 1)
    def _():
        out_ref[...] = jnp.transpose(sortedT_scr[...])


def kernel(hidden_states, W_gate, W_fc, b_fc, W_proj, b_proj):
    b, s_len, _ = hidden_states.shape
    x = hidden_states.reshape(-1, D)
    t = x.shape[0]
    num_tiles = t // LANE
    grid_sz = num_tiles + E - 1

    logits, pos, counts, aux = pl.pallas_call(
        _router_body,
        out_shape=(
            jax.ShapeDtypeStruct((t, E), jnp.float32),
            jax.ShapeDtypeStruct((t, 1), jnp.int32),
            jax.ShapeDtypeStruct((1, E), jnp.int32),
            jax.ShapeDtypeStruct((1, 1), jnp.float32),
        ),
    )(x, W_gate)

    pos1 = pos[:, 0]
    counts1 = counts[0]
    starts = (jnp.cumsum(counts1) - counts1).astype(jnp.int32)
    x_sorted = _sc_to_sorted(x, pos1)

    ends = starts + counts1
    ntiles = jnp.where(counts1 > 0,
                       (ends - 1) // LANE - starts // LANE + 1, 0)
    offs = jnp.cumsum(ntiles)
    total = offs[-1]
    gidx = jnp.arange(grid_sz, dtype=jnp.int32)
    ge = jnp.sum((offs[None, :] <= gidx[:, None]).astype(jnp.int32),
                 axis=1)
    ge = jnp.minimum(ge, E - 1)
    sel_mask = (ge[:, None] ==
                jnp.arange(E, dtype=jnp.int32)[None, :]).astype(jnp.int32)
    offs_ge = jnp.sum(sel_mask * offs[None, :], axis=1)
    ntiles_ge = jnp.sum(sel_mask * ntiles[None, :], axis=1)
    starts_ge = jnp.sum(sel_mask * starts[None, :], axis=1)
    first_of_e = offs_ge - ntiles_ge
    gt = starts_ge // LANE + (gidx - first_of_e)
    last = total - 1
    is_last = (gidx == last).astype(jnp.int32)
    ge_last = jnp.sum(is_last * ge)
    gt_last = jnp.sum(is_last * gt)
    ge = jnp.where(gidx <= last, ge, ge_last).astype(jnp.int32)
    gt = jnp.where(gidx <= last, gt, gt_last).astype(jnp.int32)

    grid_spec = pltpu.PrefetchScalarGridSpec(
        num_scalar_prefetch=4,
        grid=(grid_sz,),
        in_specs=[
            pl.BlockSpec((t, D),
                         lambda g, ge_r, gt_r, s_r, c_r: (0, 0)),
            pl.BlockSpec((DFF, D),
                         lambda g, ge_r, gt_r, s_r, c_r: (0, ge_r[g])),
            pl.BlockSpec((DFF, E),
                         lambda g, ge_r, gt_r, s_r, c_r: (0, 0)),
            pl.BlockSpec((D, DFF),
                         lambda g, ge_r, gt_r, s_r, c_r: (0, ge_r[g])),
            pl.BlockSpec((D, E),
                         lambda g, ge_r, gt_r, s_r, c_r: (0, 0)),
        ],
        out_specs=pl.BlockSpec((t, D),
                               lambda g, ge_r, gt_r, s_r, c_r: (0, 0)),
        scratch_shapes=[
            pltpu.VMEM((D, t), jnp.float32),
            pltpu.VMEM((D, t), jnp.float32),
        ],
    )
    h_rows = pl.pallas_call(
        _ffn_body,
        grid_spec=grid_spec,
        out_shape=jax.ShapeDtypeStruct((t, D), jnp.float32),
    )(ge, gt, starts, counts1, x_sorted,
      W_fc.reshape(DFF, E * D), b_fc,
      W_proj.reshape(D, E * DFF), b_proj)

    out = _sc_from_sorted(h_rows, pos1)
    return out.reshape(b, s_len, D), logits, aux[0, 0]
